# baseline (speedup 1.0000x reference)
import functools

import numpy as np

import jax
import jax.numpy as jnp
from jax import lax
from jax.experimental import pallas as pl
from jax.experimental.pallas import tpu as pltpu
from jax.experimental.pallas import tpu_sc as plsc

N = 1000000
H = 1024
W = 1024
FX = 1000.0
FY = 1000.0
CX = 512.0
CY = 512.0
TILE = 16
NTH = H // TILE
NTW = W // TILE
N_TILES = NTH * NTW
NEAR = 0.1
FAR = 100.0

PSX = 1.0 / FX
PSY = 1.0 / FY
TLX = -CX / FX
TLY = -CY / FY

_DIAG_XLA_SEGSUM = False

NP = 1 << 20
PR = 1024
PC = 1024
BR = 64
G1 = PR // BR

NC = 2
NS = 16
NW = NC * NS
GP = NP // NW
CH = 8192
NCH = GP // CH
VPC = CH // 16
ACC = N_TILES * 4


def _bf(x):
    xi = lax.bitcast_convert_type(x, jnp.int32)
    lsb = lax.shift_right_logical(xi, 16) & jnp.int32(1)
    r = (xi + jnp.int32(32767) + lsb) & jnp.int32(-65536)
    return lax.bitcast_convert_type(r, jnp.float32)


REC = np.float32(62.49999618530273)
C016 = np.float32(0.016)
CP512 = np.float32(0.512)
CM512 = np.float32(-0.512)


def _stage1_body(cw_ref, px, py, pz, qw, qx, qy, qz, s0, s1, s2,
                 cr, cg, cb, al, tid_o, wr_o, wg_o, wb_o, w_o):
    R00 = cw_ref[0]; R01 = cw_ref[1]; R02 = cw_ref[2]
    R10 = cw_ref[4]; R11 = cw_ref[5]; R12 = cw_ref[6]
    R20 = cw_ref[8]; R21 = cw_ref[9]; R22 = cw_ref[10]
    t0 = cw_ref[3]; t1 = cw_ref[7]; t2 = cw_ref[11]

    dx = _bf(px[...] - t0)
    dy = _bf(py[...] - t1)
    dz = _bf(pz[...] - t2)
    xc = dx * R00 + dy * R10 + dz * R20
    yc = dx * R01 + dy * R11 + dz * R21
    zc = dx * R02 + dy * R12 + dz * R22
    valid = (zc > NEAR) & (zc < FAR)
    one = jnp.ones_like(zc)
    zs = jnp.where(valid, zc, one)
    u = xc / zs
    v = yc / zs

    izs = one / zs
    izs2 = izs * izs
    j00 = _bf(izs)
    j02 = _bf(jnp.negative(xc) * izs2)
    j12 = _bf(jnp.negative(yc) * izs2)
    M00 = _bf(j00 * R00 + j02 * R02)
    M01 = _bf(j00 * R10 + j02 * R12)
    M02 = _bf(j00 * R20 + j02 * R22)
    M10 = _bf(j00 * R01 + j12 * R02)
    M11 = _bf(j00 * R11 + j12 * R12)
    M12 = _bf(j00 * R21 + j12 * R22)

    qwv = qw[...]; qxv = qx[...]; qyv = qy[...]; qzv = qz[...]
    qrn = one / jnp.sqrt(qwv * qwv + qxv * qxv + qyv * qyv + qzv * qzv)
    a = qwv * qrn; b = qxv * qrn; c = qyv * qrn; d = qzv * qrn
    r00 = 1.0 - (c * c + d * d) * 2.0
    r01 = (b * c - a * d) * 2.0
    r02 = (b * d + a * c) * 2.0
    r10 = (b * c + a * d) * 2.0
    r11 = 1.0 - (b * b + d * d) * 2.0
    r12 = (c * d - a * b) * 2.0
    r20 = (b * d - a * c) * 2.0
    r21 = (c * d + a * b) * 2.0
    r22 = 1.0 - (b * b + c * c) * 2.0

    e0 = jnp.exp(s0[...])
    e1 = jnp.exp(s1[...])
    e2 = jnp.exp(s2[...])

    A0 = _bf(r00 * e0); A1 = _bf(r01 * e1); A2 = _bf(r02 * e2)
    B0 = _bf(r10 * e0); B1 = _bf(r11 * e1); B2 = _bf(r12 * e2)
    C0 = _bf(r20 * e0); C1 = _bf(r21 * e1); C2 = _bf(r22 * e2)
    S00 = A0 * A0 + A1 * A1 + A2 * A2
    S01 = A0 * B0 + A1 * B1 + A2 * B2
    S02 = A0 * C0 + A1 * C1 + A2 * C2
    S11 = B0 * B0 + B1 * B1 + B2 * B2
    S12 = B0 * C0 + B1 * C1 + B2 * C2
    S22 = C0 * C0 + C1 * C1 + C2 * C2

    Sb00 = _bf(S00); Sb01 = _bf(S01); Sb02 = _bf(S02)
    Sb11 = _bf(S11); Sb12 = _bf(S12); Sb22 = _bf(S22)
    c00 = _bf(M00 * Sb00 + M01 * Sb01 + M02 * Sb02)
    c10 = _bf(M00 * Sb01 + M01 * Sb11 + M02 * Sb12)
    c20 = _bf(M00 * Sb02 + M01 * Sb12 + M02 * Sb22)
    c01 = _bf(M10 * Sb00 + M11 * Sb01 + M12 * Sb02)
    c11_ = _bf(M10 * Sb01 + M11 * Sb11 + M12 * Sb12)
    c21 = _bf(M10 * Sb02 + M11 * Sb12 + M12 * Sb22)
    cov00 = c00 * M00 + c10 * M01 + c20 * M02
    cov01 = c00 * M10 + c10 * M11 + c20 * M12
    cov10 = c01 * M00 + c11_ * M01 + c21 * M02
    cov11 = c01 * M10 + c11_ * M11 + c21 * M12
    sc00 = (cov00 + cov00) * 0.5
    sc01 = (cov01 + cov10) * 0.5
    sc11 = (cov11 + cov11) * 0.5
    det = sc00 * sc11 - sc01 * sc01

    txf = jnp.floor((u + CP512) * REC)
    tyf = jnp.floor((v + CP512) * REC)
    in_img = (txf >= 0) & (txf < NTW) & (tyf >= 0) & (tyf < NTH)
    valid = valid & in_img
    txc = jnp.clip(txf, 0.0, float(NTW - 1))
    tyc = jnp.clip(tyf, 0.0, float(NTH - 1))
    txi = txc.astype(jnp.int32)
    tyi = tyc.astype(jnp.int32)
    tid = tyi * NTW + txi

    cu = (txi.astype(jnp.float32) + 0.5) * C016 + CM512
    cv = (tyi.astype(jnp.float32) + 0.5) * C016 + CM512
    du = u - cu
    dv = v - cv
    det_c = jnp.maximum(det, 1e-12)
    idet = one / det_c
    term1 = ((sc11 * idet) * du) * du
    term2 = (((sc01 * idet) * 2.0) * du) * dv
    term3 = ((sc00 * idet) * dv) * dv
    power = ((term1 - term2) + term3) * 0.5
    g = jnp.exp(-jnp.clip(power, 0.0, 30.0))
    av = al[...]
    sig = one / (one + jnp.exp(-av))
    vf = valid.astype(jnp.float32)
    wgt = (sig * g) * vf

    tid_o[...] = tid
    wr_o[...] = cr[...] * wgt
    wg_o[...] = cg[...] * wgt
    wb_o[...] = cb[...] * wgt
    w_o[...] = wgt


def _stage1(cw16, planes):
    plane_spec = pl.BlockSpec((BR, PC), lambda i: (i, 0))
    return pl.pallas_call(
        _stage1_body,
        grid=(G1,),
        in_specs=[pl.BlockSpec(memory_space=pltpu.SMEM)] + [plane_spec] * 14,
        out_specs=[plane_spec] * 5,
        out_shape=[jax.ShapeDtypeStruct((PR, PC), jnp.int32)] +
                  [jax.ShapeDtypeStruct((PR, PC), jnp.float32)] * 4,
    )(cw16, *planes)


def _sc_body(tid_h, wr_h, wg_h, wb_h, w_h, out_h,
             tid_v, vr_v, vg_v, vb_v, vw_v,
             tid2_v, vr2_v, vg2_v, vb2_v, vw2_v, lacc_v, *sems):
    cid = lax.axis_index("c")
    sid = lax.axis_index("s")
    wid = sid * NC + cid
    base = wid * GP
    zero16 = jnp.zeros((16,), jnp.float32)

    def zbody(i, carry):
        for r in range(4):
            lacc_v[pl.ds((i * 4 + r) * 16, 16)] = zero16
        return carry
    lax.fori_loop(0, ACC // 64, zbody, 0)

    bufs = ((tid_v, vr_v, vg_v, vb_v, vw_v),
            (tid2_v, vr2_v, vg2_v, vb2_v, vw2_v))
    hbm = (tid_h, wr_h, wg_h, wb_h, w_h)

    def start(bi, k):
        off = base + k * CH
        return [pltpu.async_copy(hbm[j].at[pl.ds(off, CH)], bufs[bi][j],
                                 sems[bi]) for j in range(5)]

    def scatter(bi):
        btid, bvr, bvg, bvb, bvw = bufs[bi]

        def vbody(i, c2):
            for r in range(4):
                sl = pl.ds((i * 4 + r) * 16, 16)
                t4 = btid[sl] * 4
                plsc.addupdate_scatter(lacc_v, [t4], bvr[sl])
                plsc.addupdate_scatter(lacc_v, [t4 + 1], bvg[sl])
                plsc.addupdate_scatter(lacc_v, [t4 + 2], bvb[sl])
                plsc.addupdate_scatter(lacc_v, [t4 + 3], bvw[sl])
            return c2
        lax.fori_loop(0, VPC // 4, vbody, 0)

    handles = start(0, 0)
    for k in range(NCH):
        bi = k % 2
        nxt = start(1 - bi, k + 1) if k + 1 < NCH else None
        for h in handles:
            h.wait()
        scatter(bi)
        handles = nxt
    pltpu.sync_copy(lacc_v, out_h.at[pl.ds(wid * ACC, ACC)])


def _stage2(tid_f, wr_f, wg_f, wb_f, w_f):
    mesh = plsc.VectorSubcoreMesh(core_axis_name="c", subcore_axis_name="s")
    fn = functools.partial(
        pl.kernel, _sc_body, mesh=mesh,
        compiler_params=pltpu.CompilerParams(needs_layout_passes=False),
        out_type=jax.ShapeDtypeStruct((NW * ACC,), jnp.float32),
        scratch_types=[
            pltpu.VMEM((CH,), jnp.int32),
            pltpu.VMEM((CH,), jnp.float32),
            pltpu.VMEM((CH,), jnp.float32),
            pltpu.VMEM((CH,), jnp.float32),
            pltpu.VMEM((CH,), jnp.float32),
            pltpu.VMEM((CH,), jnp.int32),
            pltpu.VMEM((CH,), jnp.float32),
            pltpu.VMEM((CH,), jnp.float32),
            pltpu.VMEM((CH,), jnp.float32),
            pltpu.VMEM((CH,), jnp.float32),
            pltpu.VMEM((ACC,), jnp.float32),
            pltpu.SemaphoreType.DMA,
            pltpu.SemaphoreType.DMA,
        ],
    )()
    return fn(tid_f, wr_f, wg_f, wb_f, w_f)


def _stage3_body(p_ref, out_ref):
    s = jnp.sum(p_ref[...], axis=0)

    ia = lax.broadcasted_iota(jnp.int32, (256, 64), 0)
    itx = lax.broadcasted_iota(jnp.int32, (256, 64), 1)
    selw = (ia == 4 * itx + 3).astype(jnp.float32)
    wsum = jnp.dot(s, selw, preferred_element_type=jnp.float32)
    denom = wsum + 1.0

    aa = lax.broadcasted_iota(jnp.int32, (256, 192), 0)
    bb = lax.broadcasted_iota(jnp.int32, (256, 192), 1)
    selc = (aa == 4 * (bb // 3) + (bb % 3)).astype(jnp.float32)
    n3 = jnp.dot(s, selc, preferred_element_type=jnp.float32)

    tx2 = lax.broadcasted_iota(jnp.int32, (64, 192), 0)
    b2 = lax.broadcasted_iota(jnp.int32, (64, 192), 1)
    e3 = (b2 // 3 == tx2).astype(jnp.float32)
    dexp = jnp.dot(denom, e3, preferred_element_type=jnp.float32)
    t2 = n3 / dexp

    a3 = lax.broadcasted_iota(jnp.int32, (192, 3072), 0)
    l3 = lax.broadcasted_iota(jnp.int32, (192, 3072), 1)
    em = ((l3 // 48 == a3 // 3) & (l3 % 3 == a3 % 3)).astype(jnp.float32)
    t2e = jnp.dot(t2, em, preferred_element_type=jnp.float32)

    r0 = pl.program_id(0) * 128
    ri = lax.broadcasted_iota(jnp.int32, (128, 64), 0) + r0
    ty3 = lax.broadcasted_iota(jnp.int32, (128, 64), 1)
    ub = (ri // 16 == ty3).astype(jnp.float32)
    out_ref[...] = jnp.dot(ub, t2e, preferred_element_type=jnp.float32)


def _stage3(partials):
    return pl.pallas_call(
        _stage3_body,
        grid=(8,),
        in_specs=[pl.BlockSpec((NW, 64, 256), lambda i: (0, 0, 0))],
        out_specs=pl.BlockSpec((128, 3072), lambda i: (i, 0)),
        out_shape=jax.ShapeDtypeStruct((1024, 3072), jnp.float32),
    )(partials)


def kernel(mean, qvec, log_svec, color, alpha, c2w):
    big = jnp.concatenate(
        [mean.astype(jnp.float32), qvec.astype(jnp.float32),
         log_svec.astype(jnp.float32), color.astype(jnp.float32),
         alpha.astype(jnp.float32)[:, None]], axis=1)
    pad_row = jnp.array([0, 0, 0, 1, 0, 0, 0, 0, 0, 0, 0, 0, 0, -1e9],
                        jnp.float32)
    big = jnp.concatenate(
        [big, jnp.broadcast_to(pad_row, (NP - N, 14))], axis=0)
    bigT = big.T
    planes = [bigT[i].reshape(PR, PC) for i in range(14)]
    cwf = c2w.astype(jnp.float32)
    rb = _bf(cwf[:, :3])
    cw12 = jnp.concatenate([rb, cwf[:, 3:4]], axis=1).reshape(-1)
    cw16 = jnp.concatenate([cw12, jnp.zeros((4,), jnp.float32)])

    tid, wr, wg, wb, w = _stage1(cw16, planes)
    if _DIAG_XLA_SEGSUM:
        tf = tid.reshape(NP)
        accs = [jax.ops.segment_sum(v.reshape(NP), tf, num_segments=N_TILES)
                for v in (wr, wg, wb, w)]
        partials = jnp.concatenate(
            accs + [jnp.zeros((4 * N_TILES,), jnp.float32)])
    else:
        partials = _stage2(tid.reshape(NP), wr.reshape(NP), wg.reshape(NP),
                           wb.reshape(NP), w.reshape(NP))
    img = _stage3(partials.reshape(NW, 64, 256))
    return img.reshape(H, W, 3)


# speedup vs baseline: 1.9753x; 1.9753x over previous
import functools

import numpy as np

import jax
import jax.numpy as jnp
from jax import lax
from jax.experimental import pallas as pl
from jax.experimental.pallas import tpu as pltpu
from jax.experimental.pallas import tpu_sc as plsc

N = 1000000
H = 1024
W = 1024
FX = 1000.0
FY = 1000.0
CX = 512.0
CY = 512.0
TILE = 16
NTH = H // TILE
NTW = W // TILE
N_TILES = NTH * NTW
NEAR = 0.1
FAR = 100.0

PSX = 1.0 / FX
PSY = 1.0 / FY
TLX = -CX / FX
TLY = -CY / FY

_DIAG_XLA_SEGSUM = False

NP = 1 << 20
PR = 1024
PC = 1024
BR = 64
G1 = PR // BR

NC = 2
NS = 16
NW = NC * NS
GP = NP // NW
CH = 8192
NCH = GP // CH
VPC = CH // 16
ACC = N_TILES * 4


def _bf(x):
    xi = lax.bitcast_convert_type(x, jnp.int32)
    lsb = lax.shift_right_logical(xi, 16) & jnp.int32(1)
    r = (xi + jnp.int32(32767) + lsb) & jnp.int32(-65536)
    return lax.bitcast_convert_type(r, jnp.float32)


REC = np.float32(62.49999618530273)
C016 = np.float32(0.016)
CP512 = np.float32(0.512)
CM512 = np.float32(-0.512)


def _stage1_body(cw_ref, px, py, pz, qw, qx, qy, qz, s0, s1, s2,
                 cr, cg, cb, al, tid_o, wr_o, wg_o, wb_o, w_o):
    R00 = cw_ref[0]; R01 = cw_ref[1]; R02 = cw_ref[2]
    R10 = cw_ref[4]; R11 = cw_ref[5]; R12 = cw_ref[6]
    R20 = cw_ref[8]; R21 = cw_ref[9]; R22 = cw_ref[10]
    t0 = cw_ref[3]; t1 = cw_ref[7]; t2 = cw_ref[11]

    dx = _bf(px[...] - t0)
    dy = _bf(py[...] - t1)
    dz = _bf(pz[...] - t2)
    xc = dx * R00 + dy * R10 + dz * R20
    yc = dx * R01 + dy * R11 + dz * R21
    zc = dx * R02 + dy * R12 + dz * R22
    valid = (zc > NEAR) & (zc < FAR)
    one = jnp.ones_like(zc)
    zs = jnp.where(valid, zc, one)
    u = xc / zs
    v = yc / zs

    izs = one / zs
    izs2 = izs * izs
    j00 = _bf(izs)
    j02 = _bf(jnp.negative(xc) * izs2)
    j12 = _bf(jnp.negative(yc) * izs2)
    M00 = _bf(j00 * R00 + j02 * R02)
    M01 = _bf(j00 * R10 + j02 * R12)
    M02 = _bf(j00 * R20 + j02 * R22)
    M10 = _bf(j00 * R01 + j12 * R02)
    M11 = _bf(j00 * R11 + j12 * R12)
    M12 = _bf(j00 * R21 + j12 * R22)

    qwv = qw[...]; qxv = qx[...]; qyv = qy[...]; qzv = qz[...]
    qrn = one / jnp.sqrt(qwv * qwv + qxv * qxv + qyv * qyv + qzv * qzv)
    a = qwv * qrn; b = qxv * qrn; c = qyv * qrn; d = qzv * qrn
    r00 = 1.0 - (c * c + d * d) * 2.0
    r01 = (b * c - a * d) * 2.0
    r02 = (b * d + a * c) * 2.0
    r10 = (b * c + a * d) * 2.0
    r11 = 1.0 - (b * b + d * d) * 2.0
    r12 = (c * d - a * b) * 2.0
    r20 = (b * d - a * c) * 2.0
    r21 = (c * d + a * b) * 2.0
    r22 = 1.0 - (b * b + c * c) * 2.0

    e0 = jnp.exp(s0[...])
    e1 = jnp.exp(s1[...])
    e2 = jnp.exp(s2[...])

    A0 = _bf(r00 * e0); A1 = _bf(r01 * e1); A2 = _bf(r02 * e2)
    B0 = _bf(r10 * e0); B1 = _bf(r11 * e1); B2 = _bf(r12 * e2)
    C0 = _bf(r20 * e0); C1 = _bf(r21 * e1); C2 = _bf(r22 * e2)
    S00 = A0 * A0 + A1 * A1 + A2 * A2
    S01 = A0 * B0 + A1 * B1 + A2 * B2
    S02 = A0 * C0 + A1 * C1 + A2 * C2
    S11 = B0 * B0 + B1 * B1 + B2 * B2
    S12 = B0 * C0 + B1 * C1 + B2 * C2
    S22 = C0 * C0 + C1 * C1 + C2 * C2

    Sb00 = _bf(S00); Sb01 = _bf(S01); Sb02 = _bf(S02)
    Sb11 = _bf(S11); Sb12 = _bf(S12); Sb22 = _bf(S22)
    c00 = _bf(M00 * Sb00 + M01 * Sb01 + M02 * Sb02)
    c10 = _bf(M00 * Sb01 + M01 * Sb11 + M02 * Sb12)
    c20 = _bf(M00 * Sb02 + M01 * Sb12 + M02 * Sb22)
    c01 = _bf(M10 * Sb00 + M11 * Sb01 + M12 * Sb02)
    c11_ = _bf(M10 * Sb01 + M11 * Sb11 + M12 * Sb12)
    c21 = _bf(M10 * Sb02 + M11 * Sb12 + M12 * Sb22)
    cov00 = c00 * M00 + c10 * M01 + c20 * M02
    cov01 = c00 * M10 + c10 * M11 + c20 * M12
    cov10 = c01 * M00 + c11_ * M01 + c21 * M02
    cov11 = c01 * M10 + c11_ * M11 + c21 * M12
    sc00 = (cov00 + cov00) * 0.5
    sc01 = (cov01 + cov10) * 0.5
    sc11 = (cov11 + cov11) * 0.5
    det = sc00 * sc11 - sc01 * sc01

    txf = jnp.floor((u + CP512) * REC)
    tyf = jnp.floor((v + CP512) * REC)
    in_img = (txf >= 0) & (txf < NTW) & (tyf >= 0) & (tyf < NTH)
    valid = valid & in_img
    txc = jnp.clip(txf, 0.0, float(NTW - 1))
    tyc = jnp.clip(tyf, 0.0, float(NTH - 1))
    txi = txc.astype(jnp.int32)
    tyi = tyc.astype(jnp.int32)
    tid = tyi * NTW + txi

    cu = (txi.astype(jnp.float32) + 0.5) * C016 + CM512
    cv = (tyi.astype(jnp.float32) + 0.5) * C016 + CM512
    du = u - cu
    dv = v - cv
    det_c = jnp.maximum(det, 1e-12)
    idet = one / det_c
    term1 = ((sc11 * idet) * du) * du
    term2 = (((sc01 * idet) * 2.0) * du) * dv
    term3 = ((sc00 * idet) * dv) * dv
    power = ((term1 - term2) + term3) * 0.5
    g = jnp.exp(-jnp.clip(power, 0.0, 30.0))
    av = al[...]
    sig = one / (one + jnp.exp(-av))
    vf = valid.astype(jnp.float32)
    wgt = (sig * g) * vf

    tid_o[...] = tid
    wr_o[...] = cr[...] * wgt
    wg_o[...] = cg[...] * wgt
    wb_o[...] = cb[...] * wgt
    w_o[...] = wgt


def _stage1(cw16, planes):
    plane_spec = pl.BlockSpec((BR, PC), lambda i: (i, 0))
    return pl.pallas_call(
        _stage1_body,
        grid=(G1,),
        in_specs=[pl.BlockSpec(memory_space=pltpu.SMEM)] + [plane_spec] * 14,
        out_specs=[plane_spec] * 5,
        out_shape=[jax.ShapeDtypeStruct((PR, PC), jnp.int32)] +
                  [jax.ShapeDtypeStruct((PR, PC), jnp.float32)] * 4,
    )(cw16, *planes)


def _sc_body(tid_h, wr_h, wg_h, wb_h, w_h, out_h,
             tid_v, vr_v, vg_v, vb_v, vw_v,
             tid2_v, vr2_v, vg2_v, vb2_v, vw2_v, lacc_v, *sems):
    cid = lax.axis_index("c")
    sid = lax.axis_index("s")
    wid = sid * NC + cid
    base = wid * GP
    zero16 = jnp.zeros((16,), jnp.float32)

    def zbody(i, carry):
        for r in range(4):
            lacc_v[pl.ds((i * 4 + r) * 16, 16)] = zero16
        return carry
    lax.fori_loop(0, ACC // 64, zbody, 0)

    bufs = ((tid_v, vr_v, vg_v, vb_v, vw_v),
            (tid2_v, vr2_v, vg2_v, vb2_v, vw2_v))
    hbm = (tid_h, wr_h, wg_h, wb_h, w_h)

    def start(bi, k):
        off = base + k * CH
        return [pltpu.async_copy(hbm[j].at[pl.ds(off, CH)], bufs[bi][j],
                                 sems[bi]) for j in range(5)]

    def scatter(bi):
        btid, bvr, bvg, bvb, bvw = bufs[bi]

        def vbody(i, c2):
            for r in range(4):
                sl = pl.ds((i * 4 + r) * 16, 16)
                t4 = btid[sl] * 4
                plsc.addupdate_scatter(lacc_v, [t4], bvr[sl])
                plsc.addupdate_scatter(lacc_v, [t4 + 1], bvg[sl])
                plsc.addupdate_scatter(lacc_v, [t4 + 2], bvb[sl])
                plsc.addupdate_scatter(lacc_v, [t4 + 3], bvw[sl])
            return c2
        lax.fori_loop(0, VPC // 4, vbody, 0)

    handles = start(0, 0)
    for k in range(NCH):
        bi = k % 2
        nxt = start(1 - bi, k + 1) if k + 1 < NCH else None
        for h in handles:
            h.wait()
        scatter(bi)
        handles = nxt
    pltpu.sync_copy(lacc_v, out_h.at[pl.ds(wid * ACC, ACC)])


def _stage2(tid_f, wr_f, wg_f, wb_f, w_f):
    mesh = plsc.VectorSubcoreMesh(core_axis_name="c", subcore_axis_name="s")
    fn = functools.partial(
        pl.kernel, _sc_body, mesh=mesh,
        compiler_params=pltpu.CompilerParams(needs_layout_passes=False),
        out_type=jax.ShapeDtypeStruct((NW * ACC,), jnp.float32),
        scratch_types=[
            pltpu.VMEM((CH,), jnp.int32),
            pltpu.VMEM((CH,), jnp.float32),
            pltpu.VMEM((CH,), jnp.float32),
            pltpu.VMEM((CH,), jnp.float32),
            pltpu.VMEM((CH,), jnp.float32),
            pltpu.VMEM((CH,), jnp.int32),
            pltpu.VMEM((CH,), jnp.float32),
            pltpu.VMEM((CH,), jnp.float32),
            pltpu.VMEM((CH,), jnp.float32),
            pltpu.VMEM((CH,), jnp.float32),
            pltpu.VMEM((ACC,), jnp.float32),
            pltpu.SemaphoreType.DMA,
            pltpu.SemaphoreType.DMA,
        ],
    )()
    return fn(tid_f, wr_f, wg_f, wb_f, w_f)


def _stage3_body(p_ref, out_ref):
    s = jnp.sum(p_ref[...], axis=0)

    ia = lax.broadcasted_iota(jnp.int32, (256, 64), 0)
    itx = lax.broadcasted_iota(jnp.int32, (256, 64), 1)
    selw = (ia == 4 * itx + 3).astype(jnp.float32)
    wsum = jnp.dot(s, selw, preferred_element_type=jnp.float32)
    denom = wsum + 1.0

    aa = lax.broadcasted_iota(jnp.int32, (256, 192), 0)
    bb = lax.broadcasted_iota(jnp.int32, (256, 192), 1)
    selc = (aa == 4 * (bb // 3) + (bb % 3)).astype(jnp.float32)
    n3 = jnp.dot(s, selc, preferred_element_type=jnp.float32)

    tx2 = lax.broadcasted_iota(jnp.int32, (64, 192), 0)
    b2 = lax.broadcasted_iota(jnp.int32, (64, 192), 1)
    e3 = (b2 // 3 == tx2).astype(jnp.float32)
    dexp = jnp.dot(denom, e3, preferred_element_type=jnp.float32)
    t2 = n3 / dexp

    a3 = lax.broadcasted_iota(jnp.int32, (192, 3072), 0)
    l3 = lax.broadcasted_iota(jnp.int32, (192, 3072), 1)
    em = ((l3 // 48 == a3 // 3) & (l3 % 3 == a3 % 3)).astype(jnp.float32)
    t2e = jnp.dot(t2, em, preferred_element_type=jnp.float32)

    r0 = pl.program_id(0) * 128
    ri = lax.broadcasted_iota(jnp.int32, (128, 64), 0) + r0
    ty3 = lax.broadcasted_iota(jnp.int32, (128, 64), 1)
    ub = (ri // 16 == ty3).astype(jnp.float32)
    out_ref[...] = jnp.dot(ub, t2e, preferred_element_type=jnp.float32)


def _stage3(partials):
    return pl.pallas_call(
        _stage3_body,
        grid=(8,),
        in_specs=[pl.BlockSpec((NW, 64, 256), lambda i: (0, 0, 0))],
        out_specs=pl.BlockSpec((128, 3072), lambda i: (i, 0)),
        out_shape=jax.ShapeDtypeStruct((1024, 3072), jnp.float32),
    )(partials)


def _plane(v, pad_val):
    pad = jnp.full((NP - N,), pad_val, jnp.float32)
    return jnp.concatenate([v.astype(jnp.float32), pad]).reshape(PR, PC)


def kernel(mean, qvec, log_svec, color, alpha, c2w):
    planes = [
        _plane(mean[:, 0], 0.0),
        _plane(mean[:, 1], 0.0),
        _plane(mean[:, 2], 0.0),
        _plane(qvec[:, 0], 1.0),
        _plane(qvec[:, 1], 0.0),
        _plane(qvec[:, 2], 0.0),
        _plane(qvec[:, 3], 0.0),
        _plane(log_svec[:, 0], 0.0),
        _plane(log_svec[:, 1], 0.0),
        _plane(log_svec[:, 2], 0.0),
        _plane(color[:, 0], 0.0),
        _plane(color[:, 1], 0.0),
        _plane(color[:, 2], 0.0),
        _plane(alpha, -1e9),
    ]
    cwf = c2w.astype(jnp.float32)
    rb = _bf(cwf[:, :3])
    cw12 = jnp.concatenate([rb, cwf[:, 3:4]], axis=1).reshape(-1)
    cw16 = jnp.concatenate([cw12, jnp.zeros((4,), jnp.float32)])

    tid, wr, wg, wb, w = _stage1(cw16, planes)
    if _DIAG_XLA_SEGSUM:
        tf = tid.reshape(NP)
        accs = [jax.ops.segment_sum(v.reshape(NP), tf, num_segments=N_TILES)
                for v in (wr, wg, wb, w)]
        partials = jnp.concatenate(
            accs + [jnp.zeros((4 * N_TILES,), jnp.float32)])
    else:
        partials = _stage2(tid.reshape(NP), wr.reshape(NP), wg.reshape(NP),
                           wb.reshape(NP), w.reshape(NP))
    img = _stage3(partials.reshape(NW, 64, 256))
    return img.reshape(H, W, 3)


# speedup vs baseline: 1.9755x; 1.0001x over previous
import functools

import numpy as np

import jax
import jax.numpy as jnp
from jax import lax
from jax.experimental import pallas as pl
from jax.experimental.pallas import tpu as pltpu
from jax.experimental.pallas import tpu_sc as plsc

N = 1000000
H = 1024
W = 1024
FX = 1000.0
FY = 1000.0
CX = 512.0
CY = 512.0
TILE = 16
NTH = H // TILE
NTW = W // TILE
N_TILES = NTH * NTW
NEAR = 0.1
FAR = 100.0

PSX = 1.0 / FX
PSY = 1.0 / FY
TLX = -CX / FX
TLY = -CY / FY

NP = 1 << 20
PR = 1024
PC = 1024
BR = 64
G1 = PR // BR

NC = 2
NS = 16
NW = NC * NS
GP = NP // NW
CH = 8192
NCH = GP // CH
VPC = CH // 16
ACC = N_TILES * 4


def _bf(x):
    xi = lax.bitcast_convert_type(x, jnp.int32)
    lsb = lax.shift_right_logical(xi, 16) & jnp.int32(1)
    r = (xi + jnp.int32(32767) + lsb) & jnp.int32(-65536)
    return lax.bitcast_convert_type(r, jnp.float32)


REC = np.float32(62.49999618530273)
C016 = np.float32(0.016)
CP512 = np.float32(0.512)
CM512 = np.float32(-0.512)


def _stage1_body(cw_ref, px, py, pz, qw, qx, qy, qz, s0, s1, s2,
                 cr, cg, cb, al, tid_o, wr_o, wg_o, wb_o, w_o):
    R00 = cw_ref[0]; R01 = cw_ref[1]; R02 = cw_ref[2]
    R10 = cw_ref[4]; R11 = cw_ref[5]; R12 = cw_ref[6]
    R20 = cw_ref[8]; R21 = cw_ref[9]; R22 = cw_ref[10]
    t0 = cw_ref[3]; t1 = cw_ref[7]; t2 = cw_ref[11]

    dx = _bf(px[...] - t0)
    dy = _bf(py[...] - t1)
    dz = _bf(pz[...] - t2)
    xc = dx * R00 + dy * R10 + dz * R20
    yc = dx * R01 + dy * R11 + dz * R21
    zc = dx * R02 + dy * R12 + dz * R22
    valid = (zc > NEAR) & (zc < FAR)
    one = jnp.ones_like(zc)
    zs = jnp.where(valid, zc, one)
    u = xc / zs
    v = yc / zs

    izs = one / zs
    izs2 = izs * izs
    j00 = _bf(izs)
    j02 = _bf(jnp.negative(xc) * izs2)
    j12 = _bf(jnp.negative(yc) * izs2)
    M00 = _bf(j00 * R00 + j02 * R02)
    M01 = _bf(j00 * R10 + j02 * R12)
    M02 = _bf(j00 * R20 + j02 * R22)
    M10 = _bf(j00 * R01 + j12 * R02)
    M11 = _bf(j00 * R11 + j12 * R12)
    M12 = _bf(j00 * R21 + j12 * R22)

    qwv = qw[...]; qxv = qx[...]; qyv = qy[...]; qzv = qz[...]
    qrn = one / jnp.sqrt(qwv * qwv + qxv * qxv + qyv * qyv + qzv * qzv)
    a = qwv * qrn; b = qxv * qrn; c = qyv * qrn; d = qzv * qrn
    r00 = 1.0 - (c * c + d * d) * 2.0
    r01 = (b * c - a * d) * 2.0
    r02 = (b * d + a * c) * 2.0
    r10 = (b * c + a * d) * 2.0
    r11 = 1.0 - (b * b + d * d) * 2.0
    r12 = (c * d - a * b) * 2.0
    r20 = (b * d - a * c) * 2.0
    r21 = (c * d + a * b) * 2.0
    r22 = 1.0 - (b * b + c * c) * 2.0

    e0 = jnp.exp(s0[...])
    e1 = jnp.exp(s1[...])
    e2 = jnp.exp(s2[...])

    A0 = _bf(r00 * e0); A1 = _bf(r01 * e1); A2 = _bf(r02 * e2)
    B0 = _bf(r10 * e0); B1 = _bf(r11 * e1); B2 = _bf(r12 * e2)
    C0 = _bf(r20 * e0); C1 = _bf(r21 * e1); C2 = _bf(r22 * e2)
    S00 = A0 * A0 + A1 * A1 + A2 * A2
    S01 = A0 * B0 + A1 * B1 + A2 * B2
    S02 = A0 * C0 + A1 * C1 + A2 * C2
    S11 = B0 * B0 + B1 * B1 + B2 * B2
    S12 = B0 * C0 + B1 * C1 + B2 * C2
    S22 = C0 * C0 + C1 * C1 + C2 * C2

    Sb00 = _bf(S00); Sb01 = _bf(S01); Sb02 = _bf(S02)
    Sb11 = _bf(S11); Sb12 = _bf(S12); Sb22 = _bf(S22)
    c00 = _bf(M00 * Sb00 + M01 * Sb01 + M02 * Sb02)
    c10 = _bf(M00 * Sb01 + M01 * Sb11 + M02 * Sb12)
    c20 = _bf(M00 * Sb02 + M01 * Sb12 + M02 * Sb22)
    c01 = _bf(M10 * Sb00 + M11 * Sb01 + M12 * Sb02)
    c11_ = _bf(M10 * Sb01 + M11 * Sb11 + M12 * Sb12)
    c21 = _bf(M10 * Sb02 + M11 * Sb12 + M12 * Sb22)
    cov00 = c00 * M00 + c10 * M01 + c20 * M02
    cov01 = c00 * M10 + c10 * M11 + c20 * M12
    cov10 = c01 * M00 + c11_ * M01 + c21 * M02
    cov11 = c01 * M10 + c11_ * M11 + c21 * M12
    sc00 = (cov00 + cov00) * 0.5
    sc01 = (cov01 + cov10) * 0.5
    sc11 = (cov11 + cov11) * 0.5
    det = sc00 * sc11 - sc01 * sc01

    txf = jnp.floor((u + CP512) * REC)
    tyf = jnp.floor((v + CP512) * REC)
    in_img = (txf >= 0) & (txf < NTW) & (tyf >= 0) & (tyf < NTH)
    valid = valid & in_img
    txc = jnp.clip(txf, 0.0, float(NTW - 1))
    tyc = jnp.clip(tyf, 0.0, float(NTH - 1))
    txi = txc.astype(jnp.int32)
    tyi = tyc.astype(jnp.int32)
    tid = tyi * NTW + txi

    cu = (txi.astype(jnp.float32) + 0.5) * C016 + CM512
    cv = (tyi.astype(jnp.float32) + 0.5) * C016 + CM512
    du = u - cu
    dv = v - cv
    det_c = jnp.maximum(det, 1e-12)
    idet = one / det_c
    term1 = ((sc11 * idet) * du) * du
    term2 = (((sc01 * idet) * 2.0) * du) * dv
    term3 = ((sc00 * idet) * dv) * dv
    power = ((term1 - term2) + term3) * 0.5
    g = jnp.exp(-jnp.clip(power, 0.0, 30.0))
    av = al[...]
    sig = one / (one + jnp.exp(-av))
    vf = valid.astype(jnp.float32)
    wgt = (sig * g) * vf

    tid_o[...] = tid
    wr_o[...] = cr[...] * wgt
    wg_o[...] = cg[...] * wgt
    wb_o[...] = cb[...] * wgt
    w_o[...] = wgt


def _stage1(cw16, planes):
    plane_spec = pl.BlockSpec((BR, PC), lambda i: (i, 0))
    return pl.pallas_call(
        _stage1_body,
        grid=(G1,),
        in_specs=[pl.BlockSpec(memory_space=pltpu.SMEM)] + [plane_spec] * 14,
        out_specs=[plane_spec] * 5,
        out_shape=[jax.ShapeDtypeStruct((PR, PC), jnp.int32)] +
                  [jax.ShapeDtypeStruct((PR, PC), jnp.float32)] * 4,
    )(cw16, *planes)


def _sc_body(tid_h, wr_h, wg_h, wb_h, w_h, out_h,
             tid_v, vr_v, vg_v, vb_v, vw_v,
             tid2_v, vr2_v, vg2_v, vb2_v, vw2_v, lacc_v, *sems):
    cid = lax.axis_index("c")
    sid = lax.axis_index("s")
    wid = sid * NC + cid
    base = wid * GP
    zero16 = jnp.zeros((16,), jnp.float32)

    def zbody(i, carry):
        for r in range(4):
            lacc_v[pl.ds((i * 4 + r) * 16, 16)] = zero16
        return carry
    lax.fori_loop(0, ACC // 64, zbody, 0)

    bufs = ((tid_v, vr_v, vg_v, vb_v, vw_v),
            (tid2_v, vr2_v, vg2_v, vb2_v, vw2_v))
    hbm = (tid_h, wr_h, wg_h, wb_h, w_h)

    def start(bi, k):
        off = base + k * CH
        return [pltpu.async_copy(hbm[j].at[pl.ds(off, CH)], bufs[bi][j],
                                 sems[bi]) for j in range(5)]

    def scatter(bi):
        btid, bvr, bvg, bvb, bvw = bufs[bi]

        def vbody(i, c2):
            for r in range(4):
                sl = pl.ds((i * 4 + r) * 16, 16)
                t4 = btid[sl] * 4
                plsc.addupdate_scatter(lacc_v, [t4], bvr[sl])
                plsc.addupdate_scatter(lacc_v, [t4 + 1], bvg[sl])
                plsc.addupdate_scatter(lacc_v, [t4 + 2], bvb[sl])
                plsc.addupdate_scatter(lacc_v, [t4 + 3], bvw[sl])
            return c2
        lax.fori_loop(0, VPC // 4, vbody, 0)

    handles = start(0, 0)
    for k in range(NCH):
        bi = k % 2
        nxt = start(1 - bi, k + 1) if k + 1 < NCH else None
        for h in handles:
            h.wait()
        scatter(bi)
        handles = nxt
    pltpu.sync_copy(lacc_v, out_h.at[pl.ds(wid * ACC, ACC)])


def _stage2(tid_f, wr_f, wg_f, wb_f, w_f):
    mesh = plsc.VectorSubcoreMesh(core_axis_name="c", subcore_axis_name="s")
    fn = functools.partial(
        pl.kernel, _sc_body, mesh=mesh,
        compiler_params=pltpu.CompilerParams(needs_layout_passes=False),
        out_type=jax.ShapeDtypeStruct((NW * ACC,), jnp.float32),
        scratch_types=[
            pltpu.VMEM((CH,), jnp.int32),
            pltpu.VMEM((CH,), jnp.float32),
            pltpu.VMEM((CH,), jnp.float32),
            pltpu.VMEM((CH,), jnp.float32),
            pltpu.VMEM((CH,), jnp.float32),
            pltpu.VMEM((CH,), jnp.int32),
            pltpu.VMEM((CH,), jnp.float32),
            pltpu.VMEM((CH,), jnp.float32),
            pltpu.VMEM((CH,), jnp.float32),
            pltpu.VMEM((CH,), jnp.float32),
            pltpu.VMEM((ACC,), jnp.float32),
            pltpu.SemaphoreType.DMA,
            pltpu.SemaphoreType.DMA,
        ],
    )()
    return fn(tid_f, wr_f, wg_f, wb_f, w_f)


def _stage3_body(p_ref, out_ref):
    s = jnp.sum(p_ref[...], axis=0)

    ia = lax.broadcasted_iota(jnp.int32, (256, 64), 0)
    itx = lax.broadcasted_iota(jnp.int32, (256, 64), 1)
    selw = (ia == 4 * itx + 3).astype(jnp.float32)
    wsum = jnp.dot(s, selw, preferred_element_type=jnp.float32)
    denom = wsum + 1.0

    aa = lax.broadcasted_iota(jnp.int32, (256, 192), 0)
    bb = lax.broadcasted_iota(jnp.int32, (256, 192), 1)
    selc = (aa == 4 * (bb // 3) + (bb % 3)).astype(jnp.float32)
    n3 = jnp.dot(s, selc, preferred_element_type=jnp.float32)

    tx2 = lax.broadcasted_iota(jnp.int32, (64, 192), 0)
    b2 = lax.broadcasted_iota(jnp.int32, (64, 192), 1)
    e3 = (b2 // 3 == tx2).astype(jnp.float32)
    dexp = jnp.dot(denom, e3, preferred_element_type=jnp.float32)
    t2 = n3 / dexp

    a3 = lax.broadcasted_iota(jnp.int32, (192, 3072), 0)
    l3 = lax.broadcasted_iota(jnp.int32, (192, 3072), 1)
    em = ((l3 // 48 == a3 // 3) & (l3 % 3 == a3 % 3)).astype(jnp.float32)
    t2e = jnp.dot(t2, em, preferred_element_type=jnp.float32)

    r0 = pl.program_id(0) * 128
    ri = lax.broadcasted_iota(jnp.int32, (128, 64), 0) + r0
    ty3 = lax.broadcasted_iota(jnp.int32, (128, 64), 1)
    ub = (ri // 16 == ty3).astype(jnp.float32)
    out_ref[...] = jnp.dot(ub, t2e, preferred_element_type=jnp.float32)


def _stage3(partials):
    return pl.pallas_call(
        _stage3_body,
        grid=(8,),
        in_specs=[pl.BlockSpec((NW, 64, 256), lambda i: (0, 0, 0))],
        out_specs=pl.BlockSpec((128, 3072), lambda i: (i, 0)),
        out_shape=jax.ShapeDtypeStruct((1024, 3072), jnp.float32),
    )(partials)


def _plane(v, pad_val):
    pad = jnp.full((NP - N,), pad_val, jnp.float32)
    return jnp.concatenate([v.astype(jnp.float32), pad]).reshape(PR, PC)


def kernel(mean, qvec, log_svec, color, alpha, c2w):
    planes = [
        _plane(mean[:, 0], 0.0),
        _plane(mean[:, 1], 0.0),
        _plane(mean[:, 2], 0.0),
        _plane(qvec[:, 0], 1.0),
        _plane(qvec[:, 1], 0.0),
        _plane(qvec[:, 2], 0.0),
        _plane(qvec[:, 3], 0.0),
        _plane(log_svec[:, 0], 0.0),
        _plane(log_svec[:, 1], 0.0),
        _plane(log_svec[:, 2], 0.0),
        _plane(color[:, 0], 0.0),
        _plane(color[:, 1], 0.0),
        _plane(color[:, 2], 0.0),
        _plane(alpha, -1e9),
    ]
    cwf = c2w.astype(jnp.float32)
    rb = _bf(cwf[:, :3])
    cw12 = jnp.concatenate([rb, cwf[:, 3:4]], axis=1).reshape(-1)
    cw16 = jnp.concatenate([cw12, jnp.zeros((4,), jnp.float32)])

    tid, wr, wg, wb, w = _stage1(cw16, planes)
    partials = _stage2(tid.reshape(NP), wr.reshape(NP), wg.reshape(NP),
                       wb.reshape(NP), w.reshape(NP))
    img = _stage3(partials.reshape(NW, 64, 256))
    return img.reshape(H, W, 3)
